# baseline (speedup 1.0000x reference)
import functools

import jax
import jax.numpy as jnp
from jax import lax
from jax.experimental import pallas as pl
from jax.experimental.pallas import tpu as pltpu
from jax.experimental.pallas import tpu_sc as plsc

VOCAB = 100000
EMBED = 64
BATCH = 1024
CTX = 20

NUM_CORES = 2
NUM_SUBCORES = 16
LANES = 16
NW = NUM_CORES * NUM_SUBCORES
B_PER_W = BATCH // NW
IDX_PER_W = B_PER_W * CTX


def _make_gather_mean():
    mesh = plsc.VectorSubcoreMesh(core_axis_name="c", subcore_axis_name="s")

    @functools.partial(
        pl.kernel,
        mesh=mesh,
        out_type=jax.ShapeDtypeStruct((BATCH, EMBED), jnp.float32),
        compiler_params=pltpu.CompilerParams(use_tc_tiling_on_sc=False),
        scratch_types=[
            pltpu.VMEM((IDX_PER_W,), jnp.int32),
            pltpu.VMEM((IDX_PER_W, EMBED), jnp.float32),
            pltpu.VMEM((B_PER_W, EMBED), jnp.float32),
            pltpu.SemaphoreType.DMA,
        ],
    )
    def gather_mean(idx_hbm, table_hbm, out_hbm, idx_v, rows_v, pooled_v, sem):
        wid = lax.axis_index("s") * NUM_CORES + lax.axis_index("c")
        base = wid * IDX_PER_W
        pltpu.sync_copy(idx_hbm.at[pl.ds(base, IDX_PER_W)], idx_v)
        pltpu.async_copy(table_hbm.at[idx_v], rows_v, sem).wait()

        inv = jnp.float32(1.0 / CTX)

        def row_body(r, carry):
            for c in range(EMBED // LANES):
                acc = rows_v[r * CTX, pl.ds(c * LANES, LANES)]
                for t in range(1, CTX):
                    acc = acc + rows_v[r * CTX + t, pl.ds(c * LANES, LANES)]
                pooled_v[r, pl.ds(c * LANES, LANES)] = acc * inv
            return carry

        lax.fori_loop(0, B_PER_W, row_body, 0)
        pltpu.sync_copy(pooled_v, out_hbm.at[pl.ds(wid * B_PER_W, B_PER_W)])

    return gather_mean


_gather_mean_cache = []


def _gather_mean(idx_flat, emb_table):
    if not _gather_mean_cache:
        _gather_mean_cache.append(_make_gather_mean())
    return _gather_mean_cache[0](idx_flat, emb_table)


def _mm_body(x_ref, w_ref, b_ref, o_ref):
    x16 = x_ref[...].astype(jnp.bfloat16)
    w16 = w_ref[...].astype(jnp.bfloat16)
    o_ref[...] = (
        jnp.dot(x16, w16, preferred_element_type=jnp.float32) + b_ref[...]
    )


def _project(ctx_emb, W, b2d, tile_n):
    n_tiles = pl.cdiv(VOCAB, tile_n)
    return pl.pallas_call(
        _mm_body,
        grid=(n_tiles,),
        in_specs=[
            pl.BlockSpec((BATCH, EMBED), lambda j: (0, 0)),
            pl.BlockSpec((EMBED, tile_n), lambda j: (0, j)),
            pl.BlockSpec((1, tile_n), lambda j: (0, j)),
        ],
        out_specs=pl.BlockSpec((BATCH, tile_n), lambda j: (0, j)),
        out_shape=jax.ShapeDtypeStruct((BATCH, VOCAB), jnp.float32),
        compiler_params=pltpu.CompilerParams(
            dimension_semantics=("parallel",),
        ),
    )(ctx_emb, W, b2d)


def kernel(context_idxs, emb_table, W, b):
    ctx_emb = jnp.mean(jnp.take(emb_table, context_idxs, axis=0), axis=1)
    return _project(ctx_emb, W, b.reshape(1, VOCAB), tile_n=512)


# speedup vs baseline: 1.1536x; 1.1536x over previous
import functools

import jax
import jax.numpy as jnp
from jax import lax
from jax.experimental import pallas as pl
from jax.experimental.pallas import tpu as pltpu
from jax.experimental.pallas import tpu_sc as plsc

VOCAB = 100000
EMBED = 64
BATCH = 1024
CTX = 20

NUM_CORES = 2
NUM_SUBCORES = 16
LANES = 16
NW = NUM_CORES * NUM_SUBCORES
B_PER_W = BATCH // NW
IDX_PER_W = B_PER_W * CTX


def _make_gather_mean():
    mesh = plsc.VectorSubcoreMesh(core_axis_name="c", subcore_axis_name="s")

    @functools.partial(
        pl.kernel,
        mesh=mesh,
        out_type=jax.ShapeDtypeStruct((BATCH, EMBED), jnp.float32),
        compiler_params=pltpu.CompilerParams(use_tc_tiling_on_sc=False),
        scratch_types=[
            pltpu.VMEM((IDX_PER_W,), jnp.int32),
            pltpu.VMEM((IDX_PER_W, EMBED), jnp.float32),
            pltpu.VMEM((B_PER_W, EMBED), jnp.float32),
            pltpu.SemaphoreType.DMA,
        ],
    )
    def gather_mean(idx_hbm, table_hbm, out_hbm, idx_v, rows_v, pooled_v, sem):
        wid = lax.axis_index("s") * NUM_CORES + lax.axis_index("c")
        base = wid * IDX_PER_W
        pltpu.sync_copy(idx_hbm.at[pl.ds(base, IDX_PER_W)], idx_v)
        pltpu.async_copy(table_hbm.at[idx_v], rows_v, sem).wait()

        inv = jnp.float32(1.0 / CTX)

        def row_body(r, carry):
            for c in range(EMBED // LANES):
                acc = rows_v[r * CTX, pl.ds(c * LANES, LANES)]
                for t in range(1, CTX):
                    acc = acc + rows_v[r * CTX + t, pl.ds(c * LANES, LANES)]
                pooled_v[r, pl.ds(c * LANES, LANES)] = acc * inv
            return carry

        lax.fori_loop(0, B_PER_W, row_body, 0)
        pltpu.sync_copy(pooled_v, out_hbm.at[pl.ds(wid * B_PER_W, B_PER_W)])

    return gather_mean


_gather_mean_cache = []


def _gather_mean(idx_flat, emb_table):
    if not _gather_mean_cache:
        _gather_mean_cache.append(_make_gather_mean())
    return _gather_mean_cache[0](idx_flat, emb_table)


def _mm_body(x_ref, w_ref, b_ref, o_ref):
    x16 = x_ref[...].astype(jnp.bfloat16)
    w16 = w_ref[...].astype(jnp.bfloat16)
    o_ref[...] = (
        jnp.dot(x16, w16, preferred_element_type=jnp.float32) + b_ref[...]
    )


def _project(ctx_emb, W, b2d, tile_n):
    n_tiles = pl.cdiv(VOCAB, tile_n)
    return pl.pallas_call(
        _mm_body,
        grid=(n_tiles,),
        in_specs=[
            pl.BlockSpec((BATCH, EMBED), lambda j: (0, 0)),
            pl.BlockSpec((EMBED, tile_n), lambda j: (0, j)),
            pl.BlockSpec((1, tile_n), lambda j: (0, j)),
        ],
        out_specs=pl.BlockSpec((BATCH, tile_n), lambda j: (0, j)),
        out_shape=jax.ShapeDtypeStruct((BATCH, VOCAB), jnp.float32),
        compiler_params=pltpu.CompilerParams(
            dimension_semantics=("parallel",),
        ),
    )(ctx_emb, W, b2d)


def kernel(context_idxs, emb_table, W, b):
    ctx_emb = jnp.mean(jnp.take(emb_table, context_idxs, axis=0), axis=1)
    return _project(ctx_emb, W, b.reshape(1, VOCAB), tile_n=4096)
